# Initial kernel scaffold; baseline (speedup 1.0000x reference)
import jax
import jax.numpy as jnp
from jax.experimental import pallas as pl


def kernel(x, edge_index, Wl, Wr, att, bias, att_vec, att_bias, Wc, bc):
    raise NotImplementedError("write your pallas kernel here")


# speedup vs baseline: 14.3247x; 14.3247x over previous
import functools

import jax
import jax.numpy as jnp
from jax import lax
from jax.experimental import pallas as pl
from jax.experimental.pallas import tpu as pltpu
from jax.experimental.pallas import tpu_sc as plsc

N = 10000
F_IN = 128
H = 4
C = 192
HC = H * C
SUB = 64
OUT = 64

NT = 16
NCORE = 2
NTILES = NT * NCORE
W = 64
NWIN = 5
NZ = NTILES * W * NWIN

E1 = 320000 + N
E_PAD = 330240
FB = 2064
NFB = E_PAD // FB
VECS = FB // 16
GB = 16
AW = HC + 16


def _tc1_body(x_ref, wl_ref, wr_ref, xl_ref, xr_ref):
    xb = x_ref[...]
    dn = (((1,), (1,)), ((), ()))
    xl_ref[...] = lax.dot_general(xb, wl_ref[...], dn,
                                  preferred_element_type=jnp.float32)
    xr_ref[...] = lax.dot_general(xb, wr_ref[...], dn,
                                  preferred_element_type=jnp.float32)


def _tc1(x, Wl, Wr):
    br = 1000
    grid = (N // br,)
    return pl.pallas_call(
        _tc1_body,
        grid=grid,
        in_specs=[
            pl.BlockSpec((br, F_IN), lambda i: (i, jnp.int32(0))),
            pl.BlockSpec((HC, F_IN), lambda i: (jnp.int32(0), jnp.int32(0))),
            pl.BlockSpec((HC, F_IN), lambda i: (jnp.int32(0), jnp.int32(0))),
        ],
        out_specs=[
            pl.BlockSpec((br, HC), lambda i: (i, jnp.int32(0))),
            pl.BlockSpec((br, HC), lambda i: (i, jnp.int32(0))),
        ],
        out_shape=[
            jax.ShapeDtypeStruct((N, HC), jnp.float32),
            jax.ShapeDtypeStruct((N, HC), jnp.float32),
        ],
    )(x, Wl, Wr)


def _sc_body(src_h, dst_h, xl_h, xr_h, att_h, zeros_h, z_h,
             attv, srcb, dstb, fsrc, fdst, sidx, didx, gidx,
             xlb, xrb, ctb, acc, zwin, sem):
    core = lax.axis_index("c")
    tile = lax.axis_index("s")
    wid = tile * NCORE + core

    pltpu.sync_copy(att_h, attv)
    lane = lax.iota(jnp.int32, 16)
    zvec_i = jnp.zeros((16,), jnp.int32)

    def win_body(w, _):
        lo = ((wid * NWIN + w) * W).astype(jnp.int32)

        pltpu.sync_copy(zeros_h, acc)

        def fb_body(b, _, lo=lo):
            pltpu.sync_copy(src_h.at[pl.ds(b * FB, FB)], srcb)
            pltpu.sync_copy(dst_h.at[pl.ds(b * FB, FB)], dstb)

            def vec_body(k, cur, lo=lo):
                d = dstb[pl.ds(k * 16, 16)]
                sv = srcb[pl.ds(k * 16, 16)]
                m = (d >= lo) & (d < lo + W)
                mi = m.astype(jnp.int32)
                cs = jnp.cumsum(mi)
                pos = cur + cs - mi
                plsc.store_scatter(fsrc, [pos], sv, mask=m)
                plsc.store_scatter(fdst, [pos], d - lo, mask=m)
                return cur + cs[15]

            cur = lax.fori_loop(jnp.int32(0), jnp.int32(VECS), vec_body,
                                jnp.int32(0))

            for t in range(2):
                plsc.store_scatter(fsrc, [cur + t * 16 + lane], zvec_i)
                plsc.store_scatter(fdst, [cur + t * 16 + lane], zvec_i)

            nb = (cur + GB - 1) // GB

            def gb_body(g, _, lo=lo, cur=cur):
                base = g * GB
                sidx[...] = fsrc[pl.ds(base, GB)]
                dlv = fdst[pl.ds(base, GB)]
                didx[...] = dlv
                gidx[...] = dlv + lo
                pltpu.async_copy(xl_h.at[sidx], xlb, sem).wait()
                pltpu.async_copy(xr_h.at[gidx], xrb, sem).wait()

                def e_body(e, _):
                    vf = jnp.where(base + e < cur, jnp.float32(1.0), jnp.float32(0.0))
                    pbs = []
                    for h in range(H):
                        sacc = jnp.zeros((16,), jnp.float32)
                        for j in range(C // 16):
                            off = (h * (C // 16) + j) * 16
                            v = xlb[e, pl.ds(off, 16)] + xrb[e, pl.ds(off, 16)]
                            t = jnp.where(v > 0, v, v * 0.2)
                            sacc = sacc + t * attv[pl.ds(off, 16)]
                        lg = jnp.cumsum(sacc)[15]
                        pbs.append(jnp.exp(jnp.broadcast_to(lg, (16,))) * vf)
                    for h in range(H):
                        for j in range(C // 16):
                            off = (h * (C // 16) + j) * 16
                            ctb[e, pl.ds(off, 16)] = pbs[h] * xlb[e, pl.ds(off, 16)]
                    fz = jnp.float32(0.0)
                    dv = jnp.where(lane == 0, pbs[0], fz)
                    dv = dv + jnp.where(lane == 1, pbs[1], fz)
                    dv = dv + jnp.where(lane == 2, pbs[2], fz)
                    dv = dv + jnp.where(lane == 3, pbs[3], fz)
                    ctb[e, pl.ds(HC, 16)] = dv
                    return jnp.int32(0)

                lax.fori_loop(jnp.int32(0), jnp.int32(GB), e_body, jnp.int32(0))

                for e in range(GB):
                    dl_e = dlv[e]
                    for j in range(AW // 16):
                        off = j * 16
                        acc[dl_e, pl.ds(off, 16)] = (
                            acc[dl_e, pl.ds(off, 16)] + ctb[e, pl.ds(off, 16)])
                return jnp.int32(0)

            lax.fori_loop(jnp.int32(0), nb.astype(jnp.int32), gb_body,
                          jnp.int32(0))
            return jnp.int32(0)

        lax.fori_loop(jnp.int32(0), jnp.int32(NFB), fb_body, jnp.int32(0))

        def row_body(i, _):
            dvvec = acc[i, pl.ds(HC, 16)] + 1e-16
            y = plsc.bitcast(jnp.int32(0x7EF311C3)
                             - plsc.bitcast(dvvec, jnp.int32), jnp.float32)
            for _ in range(3):
                y = y * (2.0 - dvvec * y)
            y = y * (1.0 / H)
            for j in range(C // 16):
                zacc = jnp.zeros((16,), jnp.float32)
                for h in range(H):
                    off = (h * (C // 16) + j) * 16
                    zacc = zacc + acc[i, pl.ds(off, 16)] * y[h]
                zwin[i, pl.ds(j * 16, 16)] = zacc
            return jnp.int32(0)

        lax.fori_loop(jnp.int32(0), jnp.int32(W), row_body, jnp.int32(0))
        pltpu.sync_copy(zwin, z_h.at[pl.ds(lo, W)])
        return jnp.int32(0)

    lax.fori_loop(jnp.int32(0), jnp.int32(NWIN), win_body, jnp.int32(0))


@functools.partial(
    pl.kernel,
    mesh=plsc.VectorSubcoreMesh(core_axis_name="c", subcore_axis_name="s"),
    compiler_params=pltpu.CompilerParams(needs_layout_passes=False),
    out_type=jax.ShapeDtypeStruct((NZ, C), jnp.float32),
    scratch_types=[
        pltpu.VMEM((HC,), jnp.float32),
        pltpu.VMEM((FB,), jnp.int32),
        pltpu.VMEM((FB,), jnp.int32),
        pltpu.VMEM((FB + 48,), jnp.int32),
        pltpu.VMEM((FB + 48,), jnp.int32),
        pltpu.VMEM((GB,), jnp.int32),
        pltpu.VMEM((GB,), jnp.int32),
        pltpu.VMEM((GB,), jnp.int32),
        pltpu.VMEM((GB, HC), jnp.float32),
        pltpu.VMEM((GB, HC), jnp.float32),
        pltpu.VMEM((GB, AW), jnp.float32),
        pltpu.VMEM((W, AW), jnp.float32),
        pltpu.VMEM((W, C), jnp.float32),
        pltpu.SemaphoreType.DMA,
    ],
)
def _sc_edge(src_h, dst_h, xl_h, xr_h, att_h, zeros_h, z_h, *scratch):
    _sc_body(src_h, dst_h, xl_h, xr_h, att_h, zeros_h, z_h, *scratch)


def _tc2_body(z_ref, bias_ref, av_ref, ab_ref, wc_ref, bc_ref, out_ref):
    z = jnp.maximum(z_ref[...] + bias_ref[...], 0.0)
    z0 = z[:, :SUB]
    z1 = z[:, SUB:2 * SUB]
    z2 = z[:, 2 * SUB:]
    s0 = jnp.sum(z0 * av_ref[0:1, :], axis=1, keepdims=True) + ab_ref[0, 0]
    s1 = jnp.sum(z1 * av_ref[1:2, :], axis=1, keepdims=True) + ab_ref[1, 0]
    s2 = jnp.sum(z2 * av_ref[2:3, :], axis=1, keepdims=True) + ab_ref[2, 0]
    s0 = jnp.where(s0 > 0, s0, 0.01 * s0)
    s1 = jnp.where(s1 > 0, s1, 0.01 * s1)
    s2 = jnp.where(s2 > 0, s2, 0.01 * s2)
    m = jnp.maximum(jnp.maximum(s0, s1), s2)
    e0 = jnp.exp(s0 - m)
    e1 = jnp.exp(s1 - m)
    e2 = jnp.exp(s2 - m)
    den = e0 + e1 + e2
    zw = jnp.concatenate([z0 * (e0 / den), z1 * (e1 / den), z2 * (e2 / den)],
                         axis=1)
    out_ref[...] = lax.dot_general(
        zw, wc_ref[...], (((1,), (1,)), ((), ())),
        preferred_element_type=jnp.float32) + bc_ref[...]


def _tc2(z_pre, bias, att_vec, att_bias, Wc, bc):
    br = 1000
    grid = (N // br,)
    return pl.pallas_call(
        _tc2_body,
        grid=grid,
        in_specs=[
            pl.BlockSpec((br, C), lambda i: (i, jnp.int32(0))),
            pl.BlockSpec((1, C), lambda i: (jnp.int32(0), jnp.int32(0))),
            pl.BlockSpec((3, SUB), lambda i: (jnp.int32(0), jnp.int32(0))),
            pl.BlockSpec((3, 1), lambda i: (jnp.int32(0), jnp.int32(0))),
            pl.BlockSpec((OUT, C), lambda i: (jnp.int32(0), jnp.int32(0))),
            pl.BlockSpec((1, OUT), lambda i: (jnp.int32(0), jnp.int32(0))),
        ],
        out_specs=pl.BlockSpec((br, OUT), lambda i: (i, jnp.int32(0))),
        out_shape=jax.ShapeDtypeStruct((N, OUT), jnp.float32),
    )(z_pre, bias, att_vec, att_bias, Wc, bc)


def kernel(x, edge_index, Wl, Wr, att, bias, att_vec, att_bias, Wc, bc):
    x = x.astype(jnp.float32)
    ei = edge_index.astype(jnp.int32)
    ar = jnp.arange(N, dtype=jnp.int32)
    pad = E_PAD - E1
    src = jnp.concatenate([ei[0], ar, jnp.zeros((pad,), jnp.int32)])
    dst = jnp.concatenate([ei[1], ar, jnp.full((pad,), 1 << 20, jnp.int32)])

    xl, xr = _tc1(x, Wl.astype(jnp.float32), Wr.astype(jnp.float32))

    att_flat = att.astype(jnp.float32).reshape(HC)
    zeros = jnp.zeros((W, AW), jnp.float32)
    z_pre = _sc_edge(src, dst, xl, xr, att_flat, zeros)

    out = _tc2(z_pre[:N], bias.astype(jnp.float32).reshape(1, C),
               att_vec.astype(jnp.float32), att_bias.astype(jnp.float32),
               Wc.astype(jnp.float32),
               bc.astype(jnp.float32).reshape(1, OUT))
    return out.astype(jnp.float64)


# speedup vs baseline: 15.2507x; 1.0646x over previous
import functools

import jax
import jax.numpy as jnp
from jax import lax
from jax.experimental import pallas as pl
from jax.experimental.pallas import tpu as pltpu
from jax.experimental.pallas import tpu_sc as plsc

N = 10000
F_IN = 128
H = 4
C = 192
HC = H * C
SUB = 64
OUT = 64

NT = 16
NCORE = 2
NTILES = NT * NCORE
W = 64
NWIN = 5
NZ = NTILES * W * NWIN

E1 = 320000 + N
E_PAD = 330240
FB = 2064
NFB = E_PAD // FB
VECS = FB // 16
GB = 16
AW = HC + 16


def _tc1_body(x_ref, wl_ref, wr_ref, xl_ref, xr_ref):
    xb = x_ref[...]
    dn = (((1,), (1,)), ((), ()))
    xl_ref[...] = lax.dot_general(xb, wl_ref[...], dn,
                                  preferred_element_type=jnp.float32)
    xr_ref[...] = lax.dot_general(xb, wr_ref[...], dn,
                                  preferred_element_type=jnp.float32)


def _tc1(x, Wl, Wr):
    br = 1000
    grid = (N // br,)
    return pl.pallas_call(
        _tc1_body,
        grid=grid,
        in_specs=[
            pl.BlockSpec((br, F_IN), lambda i: (i, jnp.int32(0))),
            pl.BlockSpec((HC, F_IN), lambda i: (jnp.int32(0), jnp.int32(0))),
            pl.BlockSpec((HC, F_IN), lambda i: (jnp.int32(0), jnp.int32(0))),
        ],
        out_specs=[
            pl.BlockSpec((br, HC), lambda i: (i, jnp.int32(0))),
            pl.BlockSpec((br, HC), lambda i: (i, jnp.int32(0))),
        ],
        out_shape=[
            jax.ShapeDtypeStruct((N, HC), jnp.float32),
            jax.ShapeDtypeStruct((N, HC), jnp.float32),
        ],
    )(x, Wl, Wr)


def _sc_body(src_h, dst_h, xl_h, xr_h, att_h, zeros_h, z_h,
             attv, srcb, dstb, fsrc, fdst, sidx, didx, gidx,
             xlb, xrb, ctb, acc, zwin, sem, sem2):
    core = lax.axis_index("c")
    tile = lax.axis_index("s")
    wid = tile * NCORE + core

    pltpu.sync_copy(att_h, attv)
    lane = lax.iota(jnp.int32, 16)
    zvec_i = jnp.zeros((16,), jnp.int32)

    def win_body(w, _):
        lo = ((wid * NWIN + w) * W).astype(jnp.int32)

        pltpu.sync_copy(zeros_h, acc)

        def fb_body(b, _, lo=lo):
            pltpu.sync_copy(src_h.at[pl.ds(b * FB, FB)], srcb)
            pltpu.sync_copy(dst_h.at[pl.ds(b * FB, FB)], dstb)

            def vec_body(k, cur, lo=lo):
                d = dstb[pl.ds(k * 16, 16)]
                sv = srcb[pl.ds(k * 16, 16)]
                m = (d >= lo) & (d < lo + W)
                mi = m.astype(jnp.int32)
                cs = jnp.cumsum(mi)
                pos = cur + cs - mi
                plsc.store_scatter(fsrc, [pos], sv, mask=m)
                plsc.store_scatter(fdst, [pos], d - lo, mask=m)
                return cur + cs[15]

            cur = lax.fori_loop(jnp.int32(0), jnp.int32(VECS), vec_body,
                                jnp.int32(0))

            for t in range(2):
                plsc.store_scatter(fsrc, [cur + t * 16 + lane], zvec_i)
                plsc.store_scatter(fdst, [cur + t * 16 + lane], zvec_i)

            nb = (cur + GB - 1) // GB

            def gb_body(g, _, lo=lo, cur=cur):
                base = g * GB
                sidx[...] = fsrc[pl.ds(base, GB)]
                dlv = fdst[pl.ds(base, GB)]
                didx[...] = dlv
                gidx[...] = dlv + lo
                cpl = pltpu.async_copy(xl_h.at[sidx], xlb, sem)
                cpr = pltpu.async_copy(xr_h.at[gidx], xrb, sem2)
                cpl.wait()
                cpr.wait()

                def e_body(e, _):
                    vf = jnp.where(base + e < cur, jnp.float32(1.0), jnp.float32(0.0))
                    pbs = []
                    for h in range(H):
                        sacc = jnp.zeros((16,), jnp.float32)
                        for j in range(C // 16):
                            off = (h * (C // 16) + j) * 16
                            v = xlb[e, pl.ds(off, 16)] + xrb[e, pl.ds(off, 16)]
                            t = jnp.where(v > 0, v, v * 0.2)
                            sacc = sacc + t * attv[pl.ds(off, 16)]
                        lg = jnp.cumsum(sacc)[15]
                        pbs.append(jnp.exp(jnp.broadcast_to(lg, (16,))) * vf)
                    for h in range(H):
                        for j in range(C // 16):
                            off = (h * (C // 16) + j) * 16
                            ctb[e, pl.ds(off, 16)] = pbs[h] * xlb[e, pl.ds(off, 16)]
                    fz = jnp.float32(0.0)
                    dv = jnp.where(lane == 0, pbs[0], fz)
                    dv = dv + jnp.where(lane == 1, pbs[1], fz)
                    dv = dv + jnp.where(lane == 2, pbs[2], fz)
                    dv = dv + jnp.where(lane == 3, pbs[3], fz)
                    ctb[e, pl.ds(HC, 16)] = dv
                    return jnp.int32(0)

                lax.fori_loop(jnp.int32(0), jnp.int32(GB), e_body, jnp.int32(0))

                for e in range(GB):
                    dl_e = dlv[e]
                    for j in range(AW // 16):
                        off = j * 16
                        acc[dl_e, pl.ds(off, 16)] = (
                            acc[dl_e, pl.ds(off, 16)] + ctb[e, pl.ds(off, 16)])
                return jnp.int32(0)

            lax.fori_loop(jnp.int32(0), nb.astype(jnp.int32), gb_body,
                          jnp.int32(0))
            return jnp.int32(0)

        lax.fori_loop(jnp.int32(0), jnp.int32(NFB), fb_body, jnp.int32(0))

        def row_body(i, _):
            dvvec = acc[i, pl.ds(HC, 16)] + 1e-16
            y = plsc.bitcast(jnp.int32(0x7EF311C3)
                             - plsc.bitcast(dvvec, jnp.int32), jnp.float32)
            for _ in range(3):
                y = y * (2.0 - dvvec * y)
            y = y * (1.0 / H)
            for j in range(C // 16):
                zacc = jnp.zeros((16,), jnp.float32)
                for h in range(H):
                    off = (h * (C // 16) + j) * 16
                    zacc = zacc + acc[i, pl.ds(off, 16)] * y[h]
                zwin[i, pl.ds(j * 16, 16)] = zacc
            return jnp.int32(0)

        lax.fori_loop(jnp.int32(0), jnp.int32(W), row_body, jnp.int32(0))
        pltpu.sync_copy(zwin, z_h.at[pl.ds(lo, W)])
        return jnp.int32(0)

    lax.fori_loop(jnp.int32(0), jnp.int32(NWIN), win_body, jnp.int32(0))


@functools.partial(
    pl.kernel,
    mesh=plsc.VectorSubcoreMesh(core_axis_name="c", subcore_axis_name="s"),
    compiler_params=pltpu.CompilerParams(needs_layout_passes=False),
    out_type=jax.ShapeDtypeStruct((NZ, C), jnp.float32),
    scratch_types=[
        pltpu.VMEM((HC,), jnp.float32),
        pltpu.VMEM((FB,), jnp.int32),
        pltpu.VMEM((FB,), jnp.int32),
        pltpu.VMEM((FB + 48,), jnp.int32),
        pltpu.VMEM((FB + 48,), jnp.int32),
        pltpu.VMEM((GB,), jnp.int32),
        pltpu.VMEM((GB,), jnp.int32),
        pltpu.VMEM((GB,), jnp.int32),
        pltpu.VMEM((GB, HC), jnp.float32),
        pltpu.VMEM((GB, HC), jnp.float32),
        pltpu.VMEM((GB, AW), jnp.float32),
        pltpu.VMEM((W, AW), jnp.float32),
        pltpu.VMEM((W, C), jnp.float32),
        pltpu.SemaphoreType.DMA,
        pltpu.SemaphoreType.DMA,
    ],
)
def _sc_edge(src_h, dst_h, xl_h, xr_h, att_h, zeros_h, z_h, *scratch):
    _sc_body(src_h, dst_h, xl_h, xr_h, att_h, zeros_h, z_h, *scratch)


def _tc2_body(z_ref, bias_ref, av_ref, ab_ref, wc_ref, bc_ref, out_ref):
    z = jnp.maximum(z_ref[...] + bias_ref[...], 0.0)
    z0 = z[:, :SUB]
    z1 = z[:, SUB:2 * SUB]
    z2 = z[:, 2 * SUB:]
    s0 = jnp.sum(z0 * av_ref[0:1, :], axis=1, keepdims=True) + ab_ref[0, 0]
    s1 = jnp.sum(z1 * av_ref[1:2, :], axis=1, keepdims=True) + ab_ref[1, 0]
    s2 = jnp.sum(z2 * av_ref[2:3, :], axis=1, keepdims=True) + ab_ref[2, 0]
    s0 = jnp.where(s0 > 0, s0, 0.01 * s0)
    s1 = jnp.where(s1 > 0, s1, 0.01 * s1)
    s2 = jnp.where(s2 > 0, s2, 0.01 * s2)
    m = jnp.maximum(jnp.maximum(s0, s1), s2)
    e0 = jnp.exp(s0 - m)
    e1 = jnp.exp(s1 - m)
    e2 = jnp.exp(s2 - m)
    den = e0 + e1 + e2
    zw = jnp.concatenate([z0 * (e0 / den), z1 * (e1 / den), z2 * (e2 / den)],
                         axis=1)
    out_ref[...] = lax.dot_general(
        zw, wc_ref[...], (((1,), (1,)), ((), ())),
        preferred_element_type=jnp.float32) + bc_ref[...]


def _tc2(z_pre, bias, att_vec, att_bias, Wc, bc):
    br = 1000
    grid = (N // br,)
    return pl.pallas_call(
        _tc2_body,
        grid=grid,
        in_specs=[
            pl.BlockSpec((br, C), lambda i: (i, jnp.int32(0))),
            pl.BlockSpec((1, C), lambda i: (jnp.int32(0), jnp.int32(0))),
            pl.BlockSpec((3, SUB), lambda i: (jnp.int32(0), jnp.int32(0))),
            pl.BlockSpec((3, 1), lambda i: (jnp.int32(0), jnp.int32(0))),
            pl.BlockSpec((OUT, C), lambda i: (jnp.int32(0), jnp.int32(0))),
            pl.BlockSpec((1, OUT), lambda i: (jnp.int32(0), jnp.int32(0))),
        ],
        out_specs=pl.BlockSpec((br, OUT), lambda i: (i, jnp.int32(0))),
        out_shape=jax.ShapeDtypeStruct((N, OUT), jnp.float32),
    )(z_pre, bias, att_vec, att_bias, Wc, bc)


def kernel(x, edge_index, Wl, Wr, att, bias, att_vec, att_bias, Wc, bc):
    x = x.astype(jnp.float32)
    ei = edge_index.astype(jnp.int32)
    ar = jnp.arange(N, dtype=jnp.int32)
    pad = E_PAD - E1
    src = jnp.concatenate([ei[0], ar, jnp.zeros((pad,), jnp.int32)])
    dst = jnp.concatenate([ei[1], ar, jnp.full((pad,), 1 << 20, jnp.int32)])

    xl, xr = _tc1(x, Wl.astype(jnp.float32), Wr.astype(jnp.float32))

    att_flat = att.astype(jnp.float32).reshape(HC)
    zeros = jnp.zeros((W, AW), jnp.float32)
    z_pre = _sc_edge(src, dst, xl, xr, att_flat, zeros)

    out = _tc2(z_pre[:N], bias.astype(jnp.float32).reshape(1, C),
               att_vec.astype(jnp.float32), att_bias.astype(jnp.float32),
               Wc.astype(jnp.float32),
               bc.astype(jnp.float32).reshape(1, OUT))
    return out.astype(jnp.float64)


# speedup vs baseline: 25.9800x; 1.7035x over previous
import functools

import jax
import jax.numpy as jnp
from jax import lax
from jax.experimental import pallas as pl
from jax.experimental.pallas import tpu as pltpu
from jax.experimental.pallas import tpu_sc as plsc

N = 10000
F_IN = 128
H = 4
C = 192
HC = H * C
SUB = 64
OUT = 64

NT = 16
NCORE = 2
NTILES = NT * NCORE
W = 40
NWIN = 8
NZ = NTILES * W * NWIN

E1 = 320000 + N
E_PAD = 330240
FB = 2064
NFB = E_PAD // FB
VECS = FB // 16
GB = 16
AW = HC + 16


def _tc1_body(x_ref, wl_ref, wr_ref, xl_ref, xr_ref):
    xb = x_ref[...]
    dn = (((1,), (1,)), ((), ()))
    xl_ref[...] = lax.dot_general(xb, wl_ref[...], dn,
                                  preferred_element_type=jnp.float32)
    xr_ref[...] = lax.dot_general(xb, wr_ref[...], dn,
                                  preferred_element_type=jnp.float32)


def _tc1(x, Wl, Wr):
    br = 1000
    grid = (N // br,)
    return pl.pallas_call(
        _tc1_body,
        grid=grid,
        in_specs=[
            pl.BlockSpec((br, F_IN), lambda i: (i, jnp.int32(0))),
            pl.BlockSpec((HC, F_IN), lambda i: (jnp.int32(0), jnp.int32(0))),
            pl.BlockSpec((HC, F_IN), lambda i: (jnp.int32(0), jnp.int32(0))),
        ],
        out_specs=[
            pl.BlockSpec((br, HC), lambda i: (i, jnp.int32(0))),
            pl.BlockSpec((br, HC), lambda i: (i, jnp.int32(0))),
        ],
        out_shape=[
            jax.ShapeDtypeStruct((N, HC), jnp.float32),
            jax.ShapeDtypeStruct((N, HC), jnp.float32),
        ],
    )(x, Wl, Wr)


BLK = NWIN * W
CB = 2048
CROW = E_PAD + 2 * CB


def _sc_body(src_h, dst_h, xl_h, xr_h, att_h, zeros_h, z_h, coarse_h,
             attv, srcb, dstb, cflush, cload, fsrc, fdst, sidx, didx, gidx,
             xlb, xrb, ctb, acc, zwin, sem, sem2):
    core = lax.axis_index("c")
    tile = lax.axis_index("s")
    wid = tile * NCORE + core
    mybase = (wid * BLK).astype(jnp.int32)
    cbase = wid * CROW

    pltpu.sync_copy(att_h, attv)
    lane = lax.iota(jnp.int32, 16)
    zvec_i = jnp.zeros((16,), jnp.int32)
    sentinel = jnp.full((16,), -1, jnp.int32)

    def flush_if_full(carry):
        cur, mycur = carry

        def do_flush(c, m):
            pltpu.sync_copy(cflush.at[pl.ds(0, CB)],
                            coarse_h.at[pl.ds(pl.multiple_of(cbase + m, CB), CB)])

            def mv(k, _):
                cflush[pl.ds(k * 16, 16)] = cflush[pl.ds(CB + k * 16, 16)]
                return jnp.int32(0)

            nleft = c - CB
            lax.fori_loop(jnp.int32(0), (nleft + 15) // 16, mv, jnp.int32(0))
            return nleft, m + CB

        return lax.cond(cur >= CB, do_flush, lambda c, m: (c, m), cur, mycur)

    def pa_body(b, carry):
        cur, mycur = carry
        pltpu.sync_copy(src_h.at[pl.ds(b * FB, FB)], srcb)
        pltpu.sync_copy(dst_h.at[pl.ds(b * FB, FB)], dstb)

        def vec_body(k, cur):
            d = dstb[pl.ds(k * 16, 16)]
            sv = srcb[pl.ds(k * 16, 16)]
            loc = d - mybase
            m = (loc >= 0) & (loc < BLK)
            mi = m.astype(jnp.int32)
            cs = jnp.cumsum(mi)
            pos = cur + cs - mi
            plsc.store_scatter(cflush, [pos], sv * 512 + loc, mask=m)
            return cur + cs[15]

        cur = lax.fori_loop(jnp.int32(0), jnp.int32(VECS), vec_body, cur)
        return flush_if_full(flush_if_full((cur, mycur)))

    cur, mycur = lax.fori_loop(jnp.int32(0), jnp.int32(NFB), pa_body,
                               (jnp.int32(0), jnp.int32(0)))
    myc = mycur + cur

    def fill(k, _):
        plsc.store_scatter(cflush, [cur + k * 16 + lane], sentinel)
        return jnp.int32(0)

    lax.fori_loop(jnp.int32(0), (CB - cur + 15) // 16, fill, jnp.int32(0))
    pltpu.sync_copy(cflush.at[pl.ds(0, CB)],
                    coarse_h.at[pl.ds(pl.multiple_of(cbase + mycur, CB), CB)])
    nblk = (myc + CB - 1) // CB

    def win_body(w, _):
        lo = (mybase + w * W).astype(jnp.int32)
        wlo = (w * W).astype(jnp.int32)

        pltpu.sync_copy(zeros_h, acc)

        def cb_body(blk, _, lo=lo, wlo=wlo):
            pltpu.sync_copy(coarse_h.at[pl.ds(pl.multiple_of(cbase + blk * CB, CB), CB)], cload)

            def vec_body(k, cur2, wlo=wlo):
                pk = cload[pl.ds(k * 16, 16)]
                loc = pk & 511
                m = (loc >= wlo) & (loc < wlo + W)
                mi = m.astype(jnp.int32)
                cs = jnp.cumsum(mi)
                pos = cur2 + cs - mi
                plsc.store_scatter(fsrc, [pos], pk >> 9, mask=m)
                plsc.store_scatter(fdst, [pos], loc - wlo, mask=m)
                return cur2 + cs[15]

            cur2 = lax.fori_loop(jnp.int32(0), jnp.int32(CB // 16), vec_body,
                                 jnp.int32(0))

            for t in range(2):
                plsc.store_scatter(fsrc, [cur2 + t * 16 + lane], zvec_i)
                plsc.store_scatter(fdst, [cur2 + t * 16 + lane], zvec_i)

            nb = (cur2 + GB - 1) // GB

            def issue(g, lo=lo):
                ip = g & 1
                sidx[ip, :] = fsrc[pl.ds(pl.multiple_of(g * GB, GB), GB)]
                gidx[ip, :] = fdst[pl.ds(pl.multiple_of(g * GB, GB), GB)] + lo
                pltpu.async_copy(xl_h.at[sidx.at[ip]], xlb.at[ip], sem)
                pltpu.async_copy(xr_h.at[gidx.at[ip]], xrb.at[ip], sem2)

            @pl.when(nb > 0)
            def _():
                issue(jnp.int32(0))

            def gb_body(g, _, lo=lo, cur2=cur2):
                base = g * GB
                pv = g & 1
                dlv = fdst[pl.ds(pl.multiple_of(base, GB), GB)]
                pltpu.make_async_copy(xl_h.at[sidx.at[pv]], xlb.at[pv],
                                      sem).wait()
                pltpu.make_async_copy(xr_h.at[gidx.at[pv]], xrb.at[pv],
                                      sem2).wait()

                @pl.when(g + 1 < nb)
                def _():
                    issue(g + 1)

                def e_body(e, _):
                    vf = jnp.where(base + e < cur2, jnp.float32(1.0),
                                   jnp.float32(0.0))
                    pbs = []
                    for h in range(H):
                        sacc = jnp.zeros((16,), jnp.float32)
                        for j in range(C // 16):
                            off = (h * (C // 16) + j) * 16
                            v = xlb[pv, e, pl.ds(off, 16)] + xrb[pv, e, pl.ds(off, 16)]
                            t = jnp.where(v > 0, v, v * 0.2)
                            sacc = sacc + t * attv[pl.ds(off, 16)]
                        lg = jnp.cumsum(sacc)[15]
                        pbs.append(jnp.exp(jnp.broadcast_to(lg, (16,))) * vf)
                    for h in range(H):
                        for j in range(C // 16):
                            off = (h * (C // 16) + j) * 16
                            ctb[e, pl.ds(off, 16)] = pbs[h] * xlb[pv, e, pl.ds(off, 16)]
                    fz = jnp.float32(0.0)
                    dv = jnp.where(lane == 0, pbs[0], fz)
                    dv = dv + jnp.where(lane == 1, pbs[1], fz)
                    dv = dv + jnp.where(lane == 2, pbs[2], fz)
                    dv = dv + jnp.where(lane == 3, pbs[3], fz)
                    ctb[e, pl.ds(HC, 16)] = dv
                    return jnp.int32(0)

                lax.fori_loop(jnp.int32(0), jnp.int32(GB), e_body, jnp.int32(0))

                for e in range(GB):
                    dl_e = dlv[e]
                    for j in range(AW // 16):
                        off = j * 16
                        acc[dl_e, pl.ds(off, 16)] = (
                            acc[dl_e, pl.ds(off, 16)] + ctb[e, pl.ds(off, 16)])
                return jnp.int32(0)

            lax.fori_loop(jnp.int32(0), nb.astype(jnp.int32), gb_body,
                          jnp.int32(0))
            return jnp.int32(0)

        lax.fori_loop(jnp.int32(0), nblk.astype(jnp.int32), cb_body,
                      jnp.int32(0))

        def rb_body(rb, _, lo=lo):
            def row_body(ii, _):
                i = rb * 8 + ii
                dvvec = acc[i, pl.ds(HC, 16)] + 1e-16
                y = plsc.bitcast(jnp.int32(0x7EF311C3)
                                 - plsc.bitcast(dvvec, jnp.int32), jnp.float32)
                for _n in range(3):
                    y = y * (2.0 - dvvec * y)
                y = y * (1.0 / H)
                for j in range(C // 16):
                    zacc = jnp.zeros((16,), jnp.float32)
                    for h in range(H):
                        off = (h * (C // 16) + j) * 16
                        zacc = zacc + acc[i, pl.ds(off, 16)] * y[h]
                    zwin[ii, pl.ds(j * 16, 16)] = zacc
                return jnp.int32(0)

            lax.fori_loop(jnp.int32(0), jnp.int32(8), row_body, jnp.int32(0))
            pltpu.sync_copy(zwin, z_h.at[pl.ds(lo + rb * 8, 8)])
            return jnp.int32(0)

        lax.fori_loop(jnp.int32(0), jnp.int32(W // 8), rb_body, jnp.int32(0))
        return jnp.int32(0)

    lax.fori_loop(jnp.int32(0), jnp.int32(NWIN), win_body, jnp.int32(0))


@functools.partial(
    pl.kernel,
    mesh=plsc.VectorSubcoreMesh(core_axis_name="c", subcore_axis_name="s"),
    compiler_params=pltpu.CompilerParams(needs_layout_passes=False),
    out_type=(jax.ShapeDtypeStruct((NZ, C), jnp.float32),
              jax.ShapeDtypeStruct((NTILES * CROW,), jnp.int32)),
    scratch_types=[
        pltpu.VMEM((HC,), jnp.float32),
        pltpu.VMEM((FB,), jnp.int32),
        pltpu.VMEM((FB,), jnp.int32),
        pltpu.VMEM((2 * CB + FB + 32,), jnp.int32),
        pltpu.VMEM((CB,), jnp.int32),
        pltpu.VMEM((CB + 48,), jnp.int32),
        pltpu.VMEM((CB + 48,), jnp.int32),
        pltpu.VMEM((2, GB), jnp.int32),
        pltpu.VMEM((2, GB), jnp.int32),
        pltpu.VMEM((2, GB), jnp.int32),
        pltpu.VMEM((2, GB, HC), jnp.float32),
        pltpu.VMEM((2, GB, HC), jnp.float32),
        pltpu.VMEM((GB, AW), jnp.float32),
        pltpu.VMEM((W, AW), jnp.float32),
        pltpu.VMEM((8, C), jnp.float32),
        pltpu.SemaphoreType.DMA,
        pltpu.SemaphoreType.DMA,
    ],
)
def _sc_edge(src_h, dst_h, xl_h, xr_h, att_h, zeros_h, z_h, coarse_h,
             *scratch):
    _sc_body(src_h, dst_h, xl_h, xr_h, att_h, zeros_h, z_h, coarse_h,
             *scratch)


def _tc2_body(z_ref, bias_ref, av_ref, ab_ref, wc_ref, bc_ref, out_ref):
    z = jnp.maximum(z_ref[...] + bias_ref[...], 0.0)
    z0 = z[:, :SUB]
    z1 = z[:, SUB:2 * SUB]
    z2 = z[:, 2 * SUB:]
    s0 = jnp.sum(z0 * av_ref[0:1, :], axis=1, keepdims=True) + ab_ref[0, 0]
    s1 = jnp.sum(z1 * av_ref[1:2, :], axis=1, keepdims=True) + ab_ref[1, 0]
    s2 = jnp.sum(z2 * av_ref[2:3, :], axis=1, keepdims=True) + ab_ref[2, 0]
    s0 = jnp.where(s0 > 0, s0, 0.01 * s0)
    s1 = jnp.where(s1 > 0, s1, 0.01 * s1)
    s2 = jnp.where(s2 > 0, s2, 0.01 * s2)
    m = jnp.maximum(jnp.maximum(s0, s1), s2)
    e0 = jnp.exp(s0 - m)
    e1 = jnp.exp(s1 - m)
    e2 = jnp.exp(s2 - m)
    den = e0 + e1 + e2
    zw = jnp.concatenate([z0 * (e0 / den), z1 * (e1 / den), z2 * (e2 / den)],
                         axis=1)
    out_ref[...] = lax.dot_general(
        zw, wc_ref[...], (((1,), (1,)), ((), ())),
        preferred_element_type=jnp.float32) + bc_ref[...]


def _tc2(z_pre, bias, att_vec, att_bias, Wc, bc):
    br = 1000
    grid = (N // br,)
    return pl.pallas_call(
        _tc2_body,
        grid=grid,
        in_specs=[
            pl.BlockSpec((br, C), lambda i: (i, jnp.int32(0))),
            pl.BlockSpec((1, C), lambda i: (jnp.int32(0), jnp.int32(0))),
            pl.BlockSpec((3, SUB), lambda i: (jnp.int32(0), jnp.int32(0))),
            pl.BlockSpec((3, 1), lambda i: (jnp.int32(0), jnp.int32(0))),
            pl.BlockSpec((OUT, C), lambda i: (jnp.int32(0), jnp.int32(0))),
            pl.BlockSpec((1, OUT), lambda i: (jnp.int32(0), jnp.int32(0))),
        ],
        out_specs=pl.BlockSpec((br, OUT), lambda i: (i, jnp.int32(0))),
        out_shape=jax.ShapeDtypeStruct((N, OUT), jnp.float32),
    )(z_pre, bias, att_vec, att_bias, Wc, bc)


def kernel(x, edge_index, Wl, Wr, att, bias, att_vec, att_bias, Wc, bc):
    x = x.astype(jnp.float32)
    ei = edge_index.astype(jnp.int32)
    ar = jnp.arange(N, dtype=jnp.int32)
    pad = E_PAD - E1
    src = jnp.concatenate([ei[0], ar, jnp.zeros((pad,), jnp.int32)])
    dst = jnp.concatenate([ei[1], ar, jnp.full((pad,), 1 << 20, jnp.int32)])

    xl, xr = _tc1(x, Wl.astype(jnp.float32), Wr.astype(jnp.float32))

    att_flat = att.astype(jnp.float32).reshape(HC)
    zeros = jnp.zeros((W, AW), jnp.float32)
    z_pre, _ = _sc_edge(src, dst, xl, xr, att_flat, zeros)

    out = _tc2(z_pre[:N], bias.astype(jnp.float32).reshape(1, C),
               att_vec.astype(jnp.float32), att_bias.astype(jnp.float32),
               Wc.astype(jnp.float32),
               bc.astype(jnp.float32).reshape(1, OUT))
    return out.astype(jnp.float64)


# speedup vs baseline: 38.9982x; 1.5011x over previous
import functools

import jax
import jax.numpy as jnp
from jax import lax
from jax.experimental import pallas as pl
from jax.experimental.pallas import tpu as pltpu
from jax.experimental.pallas import tpu_sc as plsc

N = 10000
F_IN = 128
H = 4
C = 192
HC = H * C
SUB = 64
OUT = 64

NT = 16
NCORE = 2
NTILES = NT * NCORE
W = 40
NWIN = 8
NZ = NTILES * W * NWIN

E1 = 320000 + N
E_PAD = 330240
FB = 2064
NFB = E_PAD // FB
VECS = FB // 16
GB = 16
AW = HC + 16


def _tc1_body(x_ref, wl_ref, wr_ref, xl_ref, xr_ref):
    xb = x_ref[...]
    dn = (((1,), (1,)), ((), ()))
    xl_ref[...] = lax.dot_general(xb, wl_ref[...], dn,
                                  preferred_element_type=jnp.float32)
    xr_ref[...] = lax.dot_general(xb, wr_ref[...], dn,
                                  preferred_element_type=jnp.float32)


def _tc1(x, Wl, Wr):
    br = 1000
    grid = (N // br,)
    return pl.pallas_call(
        _tc1_body,
        grid=grid,
        in_specs=[
            pl.BlockSpec((br, F_IN), lambda i: (i, jnp.int32(0))),
            pl.BlockSpec((HC, F_IN), lambda i: (jnp.int32(0), jnp.int32(0))),
            pl.BlockSpec((HC, F_IN), lambda i: (jnp.int32(0), jnp.int32(0))),
        ],
        out_specs=[
            pl.BlockSpec((br, HC), lambda i: (i, jnp.int32(0))),
            pl.BlockSpec((br, HC), lambda i: (i, jnp.int32(0))),
        ],
        out_shape=[
            jax.ShapeDtypeStruct((N, HC), jnp.float32),
            jax.ShapeDtypeStruct((N, HC), jnp.float32),
        ],
    )(x, Wl, Wr)


BLK = NWIN * W
CB = 2048
CROW = E_PAD + 2 * CB


def _sc_body(src_h, dst_h, xl_h, xr_h, att_h, zeros_h, z_h, coarse_h,
             attv, srcb, dstb, cflush, cload, fsrc, fdst, sidx, didx, gidx,
             xlb, xrb, acc, zwin, sem, sem2):
    core = lax.axis_index("c")
    tile = lax.axis_index("s")
    wid = tile * NCORE + core
    mybase = (wid * BLK).astype(jnp.int32)
    cbase = wid * CROW

    pltpu.sync_copy(att_h, attv)
    lane = lax.iota(jnp.int32, 16)
    zvec_i = jnp.zeros((16,), jnp.int32)
    sentinel = jnp.full((16,), -1, jnp.int32)

    def flush_if_full(carry):
        cur, mycur = carry

        def do_flush(c, m):
            pltpu.sync_copy(cflush.at[pl.ds(0, CB)],
                            coarse_h.at[pl.ds(pl.multiple_of(cbase + m, CB), CB)])

            def mv(k, _):
                cflush[pl.ds(k * 16, 16)] = cflush[pl.ds(CB + k * 16, 16)]
                return jnp.int32(0)

            nleft = c - CB
            lax.fori_loop(jnp.int32(0), (nleft + 15) // 16, mv, jnp.int32(0))
            return nleft, m + CB

        return lax.cond(cur >= CB, do_flush, lambda c, m: (c, m), cur, mycur)

    def pa_body(b, carry):
        cur, mycur = carry
        pltpu.sync_copy(src_h.at[pl.ds(b * FB, FB)], srcb)
        pltpu.sync_copy(dst_h.at[pl.ds(b * FB, FB)], dstb)

        def vec_body(k, cur):
            d = dstb[pl.ds(k * 16, 16)]
            sv = srcb[pl.ds(k * 16, 16)]
            loc = d - mybase
            m = (loc >= 0) & (loc < BLK)
            mi = m.astype(jnp.int32)
            cs = jnp.cumsum(mi)
            pos = cur + cs - mi
            plsc.store_scatter(cflush, [pos], sv * 512 + loc, mask=m)
            return cur + cs[15]

        cur = lax.fori_loop(jnp.int32(0), jnp.int32(VECS), vec_body, cur)
        return flush_if_full(flush_if_full((cur, mycur)))

    cur, mycur = lax.fori_loop(jnp.int32(0), jnp.int32(NFB), pa_body,
                               (jnp.int32(0), jnp.int32(0)))
    myc = mycur + cur

    def fill(k, _):
        plsc.store_scatter(cflush, [cur + k * 16 + lane], sentinel)
        return jnp.int32(0)

    lax.fori_loop(jnp.int32(0), (CB - cur + 15) // 16, fill, jnp.int32(0))
    pltpu.sync_copy(cflush.at[pl.ds(0, CB)],
                    coarse_h.at[pl.ds(pl.multiple_of(cbase + mycur, CB), CB)])
    nblk = (myc + CB - 1) // CB

    def win_body(w, _):
        lo = (mybase + w * W).astype(jnp.int32)
        wlo = (w * W).astype(jnp.int32)

        pltpu.sync_copy(zeros_h, acc)

        def cb_body(blk, _, lo=lo, wlo=wlo):
            pltpu.sync_copy(coarse_h.at[pl.ds(pl.multiple_of(cbase + blk * CB, CB), CB)], cload)

            def vec_body(k, cur2, wlo=wlo):
                pk = cload[pl.ds(k * 16, 16)]
                loc = pk & 511
                m = (loc >= wlo) & (loc < wlo + W)
                mi = m.astype(jnp.int32)
                cs = jnp.cumsum(mi)
                pos = cur2 + cs - mi
                plsc.store_scatter(fsrc, [pos], pk >> 9, mask=m)
                plsc.store_scatter(fdst, [pos], loc - wlo, mask=m)
                return cur2 + cs[15]

            cur2 = lax.fori_loop(jnp.int32(0), jnp.int32(CB // 16), vec_body,
                                 jnp.int32(0))

            for t in range(2):
                plsc.store_scatter(fsrc, [cur2 + t * 16 + lane], zvec_i)
                plsc.store_scatter(fdst, [cur2 + t * 16 + lane], zvec_i)

            nb = (cur2 + GB - 1) // GB

            def issue(g, lo=lo):
                ip = g & 1
                sidx[ip, :] = fsrc[pl.ds(pl.multiple_of(g * GB, GB), GB)]
                gidx[ip, :] = fdst[pl.ds(pl.multiple_of(g * GB, GB), GB)] + lo
                pltpu.async_copy(xl_h.at[sidx.at[ip]], xlb.at[ip], sem)
                pltpu.async_copy(xr_h.at[gidx.at[ip]], xrb.at[ip], sem2)

            @pl.when(nb > 0)
            def _():
                issue(jnp.int32(0))

            def gb_body(g, _, lo=lo, cur2=cur2):
                base = g * GB
                pv = g & 1
                dlv = fdst[pl.ds(pl.multiple_of(base, GB), GB)]
                pltpu.make_async_copy(xl_h.at[sidx.at[pv]], xlb.at[pv],
                                      sem).wait()
                pltpu.make_async_copy(xr_h.at[gidx.at[pv]], xrb.at[pv],
                                      sem2).wait()

                @pl.when(g + 1 < nb)
                def _():
                    issue(g + 1)

                def e_body(e, _):
                    vf = jnp.where(base + e < cur2, jnp.float32(1.0),
                                   jnp.float32(0.0))
                    ev = jnp.broadcast_to(e, (16,))
                    dlbc = dlv.at[ev].get(mode="promise_in_bounds")
                    pbs = []
                    for h in range(H):
                        sacc = jnp.zeros((16,), jnp.float32)
                        for j in range(C // 16):
                            off = (h * (C // 16) + j) * 16
                            v = xlb[pv, e, pl.ds(off, 16)] + xrb[pv, e, pl.ds(off, 16)]
                            t = jnp.where(v > 0, v, v * 0.2)
                            sacc = sacc + t * attv[pl.ds(off, 16)]
                        for sh in (8, 4, 2, 1):
                            sacc = sacc + sacc.at[(lane + sh) & 15].get(
                                mode="promise_in_bounds")
                        pbs.append(jnp.exp(sacc) * vf)
                    for h in range(H):
                        for j in range(C // 16):
                            off = (h * (C // 16) + j) * 16
                            plsc.addupdate_scatter(
                                acc, [dlbc, lane + off],
                                pbs[h] * xlb[pv, e, pl.ds(off, 16)])
                    fz = jnp.float32(0.0)
                    dv = jnp.where(lane == 0, pbs[0], fz)
                    dv = dv + jnp.where(lane == 1, pbs[1], fz)
                    dv = dv + jnp.where(lane == 2, pbs[2], fz)
                    dv = dv + jnp.where(lane == 3, pbs[3], fz)
                    plsc.addupdate_scatter(acc, [dlbc, lane + HC], dv)
                    return jnp.int32(0)

                lax.fori_loop(jnp.int32(0), jnp.int32(GB), e_body, jnp.int32(0))
                return jnp.int32(0)

            lax.fori_loop(jnp.int32(0), nb.astype(jnp.int32), gb_body,
                          jnp.int32(0))
            return jnp.int32(0)

        lax.fori_loop(jnp.int32(0), nblk.astype(jnp.int32), cb_body,
                      jnp.int32(0))

        def rb_body(rb, _, lo=lo):
            def row_body(ii, _):
                i = rb * 8 + ii
                dvvec = acc[i, pl.ds(HC, 16)] + 1e-16
                y = plsc.bitcast(jnp.int32(0x7EF311C3)
                                 - plsc.bitcast(dvvec, jnp.int32), jnp.float32)
                for _n in range(3):
                    y = y * (2.0 - dvvec * y)
                y = y * (1.0 / H)
                for j in range(C // 16):
                    zacc = jnp.zeros((16,), jnp.float32)
                    for h in range(H):
                        off = (h * (C // 16) + j) * 16
                        zacc = zacc + acc[i, pl.ds(off, 16)] * y[h]
                    zwin[ii, pl.ds(j * 16, 16)] = zacc
                return jnp.int32(0)

            lax.fori_loop(jnp.int32(0), jnp.int32(8), row_body, jnp.int32(0))
            pltpu.sync_copy(zwin, z_h.at[pl.ds(lo + rb * 8, 8)])
            return jnp.int32(0)

        lax.fori_loop(jnp.int32(0), jnp.int32(W // 8), rb_body, jnp.int32(0))
        return jnp.int32(0)

    lax.fori_loop(jnp.int32(0), jnp.int32(NWIN), win_body, jnp.int32(0))


@functools.partial(
    pl.kernel,
    mesh=plsc.VectorSubcoreMesh(core_axis_name="c", subcore_axis_name="s"),
    compiler_params=pltpu.CompilerParams(needs_layout_passes=False),
    out_type=(jax.ShapeDtypeStruct((NZ, C), jnp.float32),
              jax.ShapeDtypeStruct((NTILES * CROW,), jnp.int32)),
    scratch_types=[
        pltpu.VMEM((HC,), jnp.float32),
        pltpu.VMEM((FB,), jnp.int32),
        pltpu.VMEM((FB,), jnp.int32),
        pltpu.VMEM((2 * CB + FB + 32,), jnp.int32),
        pltpu.VMEM((CB,), jnp.int32),
        pltpu.VMEM((CB + 48,), jnp.int32),
        pltpu.VMEM((CB + 48,), jnp.int32),
        pltpu.VMEM((2, GB), jnp.int32),
        pltpu.VMEM((2, GB), jnp.int32),
        pltpu.VMEM((2, GB), jnp.int32),
        pltpu.VMEM((2, GB, HC), jnp.float32),
        pltpu.VMEM((2, GB, HC), jnp.float32),
        pltpu.VMEM((W, AW), jnp.float32),
        pltpu.VMEM((8, C), jnp.float32),
        pltpu.SemaphoreType.DMA,
        pltpu.SemaphoreType.DMA,
    ],
)
def _sc_edge(src_h, dst_h, xl_h, xr_h, att_h, zeros_h, z_h, coarse_h,
             *scratch):
    _sc_body(src_h, dst_h, xl_h, xr_h, att_h, zeros_h, z_h, coarse_h,
             *scratch)


def _tc2_body(z_ref, bias_ref, av_ref, ab_ref, wc_ref, bc_ref, out_ref):
    z = jnp.maximum(z_ref[...] + bias_ref[...], 0.0)
    z0 = z[:, :SUB]
    z1 = z[:, SUB:2 * SUB]
    z2 = z[:, 2 * SUB:]
    s0 = jnp.sum(z0 * av_ref[0:1, :], axis=1, keepdims=True) + ab_ref[0, 0]
    s1 = jnp.sum(z1 * av_ref[1:2, :], axis=1, keepdims=True) + ab_ref[1, 0]
    s2 = jnp.sum(z2 * av_ref[2:3, :], axis=1, keepdims=True) + ab_ref[2, 0]
    s0 = jnp.where(s0 > 0, s0, 0.01 * s0)
    s1 = jnp.where(s1 > 0, s1, 0.01 * s1)
    s2 = jnp.where(s2 > 0, s2, 0.01 * s2)
    m = jnp.maximum(jnp.maximum(s0, s1), s2)
    e0 = jnp.exp(s0 - m)
    e1 = jnp.exp(s1 - m)
    e2 = jnp.exp(s2 - m)
    den = e0 + e1 + e2
    zw = jnp.concatenate([z0 * (e0 / den), z1 * (e1 / den), z2 * (e2 / den)],
                         axis=1)
    out_ref[...] = lax.dot_general(
        zw, wc_ref[...], (((1,), (1,)), ((), ())),
        preferred_element_type=jnp.float32) + bc_ref[...]


def _tc2(z_pre, bias, att_vec, att_bias, Wc, bc):
    br = 1000
    grid = (N // br,)
    return pl.pallas_call(
        _tc2_body,
        grid=grid,
        in_specs=[
            pl.BlockSpec((br, C), lambda i: (i, jnp.int32(0))),
            pl.BlockSpec((1, C), lambda i: (jnp.int32(0), jnp.int32(0))),
            pl.BlockSpec((3, SUB), lambda i: (jnp.int32(0), jnp.int32(0))),
            pl.BlockSpec((3, 1), lambda i: (jnp.int32(0), jnp.int32(0))),
            pl.BlockSpec((OUT, C), lambda i: (jnp.int32(0), jnp.int32(0))),
            pl.BlockSpec((1, OUT), lambda i: (jnp.int32(0), jnp.int32(0))),
        ],
        out_specs=pl.BlockSpec((br, OUT), lambda i: (i, jnp.int32(0))),
        out_shape=jax.ShapeDtypeStruct((N, OUT), jnp.float32),
    )(z_pre, bias, att_vec, att_bias, Wc, bc)


def kernel(x, edge_index, Wl, Wr, att, bias, att_vec, att_bias, Wc, bc):
    x = x.astype(jnp.float32)
    ei = edge_index.astype(jnp.int32)
    ar = jnp.arange(N, dtype=jnp.int32)
    pad = E_PAD - E1
    src = jnp.concatenate([ei[0], ar, jnp.zeros((pad,), jnp.int32)])
    dst = jnp.concatenate([ei[1], ar, jnp.full((pad,), 1 << 20, jnp.int32)])

    xl, xr = _tc1(x, Wl.astype(jnp.float32), Wr.astype(jnp.float32))

    att_flat = att.astype(jnp.float32).reshape(HC)
    zeros = jnp.zeros((W, AW), jnp.float32)
    z_pre, _ = _sc_edge(src, dst, xl, xr, att_flat, zeros)

    out = _tc2(z_pre[:N], bias.astype(jnp.float32).reshape(1, C),
               att_vec.astype(jnp.float32), att_bias.astype(jnp.float32),
               Wc.astype(jnp.float32),
               bc.astype(jnp.float32).reshape(1, OUT))
    return out.astype(jnp.float64)


# speedup vs baseline: 40.3691x; 1.0352x over previous
import functools

import jax
import jax.numpy as jnp
from jax import lax
from jax.experimental import pallas as pl
from jax.experimental.pallas import tpu as pltpu
from jax.experimental.pallas import tpu_sc as plsc

N = 10000
F_IN = 128
H = 4
C = 192
HC = H * C
SUB = 64
OUT = 64

NT = 16
NCORE = 2
NTILES = NT * NCORE
W = 40
NWIN = 8
NZ = NTILES * W * NWIN

E1 = 320000 + N
E_PAD = 330240
FB = 2064
NFB = E_PAD // FB
VECS = FB // 16
GB = 16
AW = HC + 16


def _tc1_body(x_ref, wl_ref, wr_ref, xl_ref, xr_ref):
    xb = x_ref[...]
    dn = (((1,), (1,)), ((), ()))
    xl_ref[...] = lax.dot_general(xb, wl_ref[...], dn,
                                  preferred_element_type=jnp.float32)
    xr_ref[...] = lax.dot_general(xb, wr_ref[...], dn,
                                  preferred_element_type=jnp.float32)


def _tc1(x, Wl, Wr):
    br = 1000
    grid = (N // br,)
    return pl.pallas_call(
        _tc1_body,
        grid=grid,
        in_specs=[
            pl.BlockSpec((br, F_IN), lambda i: (i, jnp.int32(0))),
            pl.BlockSpec((HC, F_IN), lambda i: (jnp.int32(0), jnp.int32(0))),
            pl.BlockSpec((HC, F_IN), lambda i: (jnp.int32(0), jnp.int32(0))),
        ],
        out_specs=[
            pl.BlockSpec((br, HC), lambda i: (i, jnp.int32(0))),
            pl.BlockSpec((br, HC), lambda i: (i, jnp.int32(0))),
        ],
        out_shape=[
            jax.ShapeDtypeStruct((N, HC), jnp.float32),
            jax.ShapeDtypeStruct((N, HC), jnp.float32),
        ],
    )(x, Wl, Wr)


BLK = NWIN * W
CB = 2048
CROW = E_PAD + 2 * CB


def _sc_body(src_h, dst_h, xl_h, xr_h, att_h, zeros_h, z_h, coarse_h,
             attv, srcb, dstb, cflush, cload, fsrc, fdst, sidx, didx, gidx,
             xlb, xrb, acc, zwin, sem, sem2):
    core = lax.axis_index("c")
    tile = lax.axis_index("s")
    wid = tile * NCORE + core
    mybase = (wid * BLK).astype(jnp.int32)
    cbase = wid * CROW

    pltpu.sync_copy(att_h, attv)
    lane = lax.iota(jnp.int32, 16)
    zvec_i = jnp.zeros((16,), jnp.int32)
    sentinel = jnp.full((16,), -1, jnp.int32)

    def prefix16(mi):
        v = mi
        for sh in (1, 2, 4, 8):
            sh_v = v.at[(lane - sh) & 15].get(mode="promise_in_bounds")
            v = v + jnp.where(lane >= sh, sh_v, jnp.int32(0))
        return v

    def flush_if_full(carry):
        cur, mycur = carry

        def do_flush(c, m):
            pltpu.sync_copy(cflush.at[pl.ds(0, CB)],
                            coarse_h.at[pl.ds(pl.multiple_of(cbase + m, CB), CB)])

            def mv(k, _):
                cflush[pl.ds(k * 16, 16)] = cflush[pl.ds(CB + k * 16, 16)]
                return jnp.int32(0)

            nleft = c - CB
            lax.fori_loop(jnp.int32(0), (nleft + 15) // 16, mv, jnp.int32(0))
            return nleft, m + CB

        return lax.cond(cur >= CB, do_flush, lambda c, m: (c, m), cur, mycur)

    def pa_body(b, carry):
        cur, mycur = carry
        c1 = pltpu.async_copy(src_h.at[pl.ds(b * FB, FB)], srcb, sem)
        c2 = pltpu.async_copy(dst_h.at[pl.ds(b * FB, FB)], dstb, sem2)
        c1.wait()
        c2.wait()

        def vec_body(k, cur):
            d = dstb[pl.ds(k * 16, 16)]
            sv = srcb[pl.ds(k * 16, 16)]
            loc = d - mybase
            m = (loc >= 0) & (loc < BLK)
            mi = m.astype(jnp.int32)
            cs = prefix16(mi)
            pos = cur + cs - mi
            plsc.store_scatter(cflush, [pos], sv * 512 + loc, mask=m)
            return cur + cs[15]

        cur = lax.fori_loop(jnp.int32(0), jnp.int32(VECS), vec_body, cur)
        return flush_if_full(flush_if_full((cur, mycur)))

    cur, mycur = lax.fori_loop(jnp.int32(0), jnp.int32(NFB), pa_body,
                               (jnp.int32(0), jnp.int32(0)))
    myc = mycur + cur

    def fill(k, _):
        plsc.store_scatter(cflush, [cur + k * 16 + lane], sentinel)
        return jnp.int32(0)

    lax.fori_loop(jnp.int32(0), (CB - cur + 15) // 16, fill, jnp.int32(0))
    pltpu.sync_copy(cflush.at[pl.ds(0, CB)],
                    coarse_h.at[pl.ds(pl.multiple_of(cbase + mycur, CB), CB)])
    nblk = (myc + CB - 1) // CB

    def win_body(w, _):
        lo = (mybase + w * W).astype(jnp.int32)
        wlo = (w * W).astype(jnp.int32)

        pltpu.sync_copy(zeros_h, acc)

        def cb_body(blk, _, lo=lo, wlo=wlo):
            pltpu.sync_copy(coarse_h.at[pl.ds(pl.multiple_of(cbase + blk * CB, CB), CB)], cload)

            def vec_body(k, cur2, wlo=wlo):
                pk = cload[pl.ds(k * 16, 16)]
                loc = pk & 511
                m = (loc >= wlo) & (loc < wlo + W)
                mi = m.astype(jnp.int32)
                cs = prefix16(mi)
                pos = cur2 + cs - mi
                plsc.store_scatter(fsrc, [pos], pk >> 9, mask=m)
                plsc.store_scatter(fdst, [pos], loc - wlo, mask=m)
                return cur2 + cs[15]

            cur2 = lax.fori_loop(jnp.int32(0), jnp.int32(CB // 16), vec_body,
                                 jnp.int32(0))

            for t in range(2):
                plsc.store_scatter(fsrc, [cur2 + t * 16 + lane], zvec_i)
                plsc.store_scatter(fdst, [cur2 + t * 16 + lane], zvec_i)

            nb = (cur2 + GB - 1) // GB

            def issue(g, lo=lo):
                ip = g & 1
                sidx[ip, :] = fsrc[pl.ds(pl.multiple_of(g * GB, GB), GB)]
                gidx[ip, :] = fdst[pl.ds(pl.multiple_of(g * GB, GB), GB)] + lo
                pltpu.async_copy(xl_h.at[sidx.at[ip]], xlb.at[ip], sem)
                pltpu.async_copy(xr_h.at[gidx.at[ip]], xrb.at[ip], sem2)

            @pl.when(nb > 0)
            def _():
                issue(jnp.int32(0))

            def gb_body(g, _, lo=lo, cur2=cur2):
                base = g * GB
                pv = g & 1
                dlv = fdst[pl.ds(pl.multiple_of(base, GB), GB)]
                pltpu.make_async_copy(xl_h.at[sidx.at[pv]], xlb.at[pv],
                                      sem).wait()
                pltpu.make_async_copy(xr_h.at[gidx.at[pv]], xrb.at[pv],
                                      sem2).wait()

                @pl.when(g + 1 < nb)
                def _():
                    issue(g + 1)

                def e_body(e, _):
                    vf = jnp.where(base + e < cur2, jnp.float32(1.0),
                                   jnp.float32(0.0))
                    ev = jnp.broadcast_to(e, (16,))
                    dlbc = dlv.at[ev].get(mode="promise_in_bounds")
                    pbs = []
                    for h in range(H):
                        sacc = jnp.zeros((16,), jnp.float32)
                        for j in range(C // 16):
                            off = (h * (C // 16) + j) * 16
                            v = xlb[pv, e, pl.ds(off, 16)] + xrb[pv, e, pl.ds(off, 16)]
                            t = jnp.where(v > 0, v, v * 0.2)
                            sacc = sacc + t * attv[pl.ds(off, 16)]
                        for sh in (8, 4, 2, 1):
                            sacc = sacc + sacc.at[(lane + sh) & 15].get(
                                mode="promise_in_bounds")
                        pbs.append(jnp.exp(sacc) * vf)
                    for h in range(H):
                        for j in range(C // 16):
                            off = (h * (C // 16) + j) * 16
                            plsc.addupdate_scatter(
                                acc, [dlbc, lane + off],
                                pbs[h] * xlb[pv, e, pl.ds(off, 16)])
                    fz = jnp.float32(0.0)
                    dv = jnp.where(lane == 0, pbs[0], fz)
                    dv = dv + jnp.where(lane == 1, pbs[1], fz)
                    dv = dv + jnp.where(lane == 2, pbs[2], fz)
                    dv = dv + jnp.where(lane == 3, pbs[3], fz)
                    plsc.addupdate_scatter(acc, [dlbc, lane + HC], dv)
                    return jnp.int32(0)

                lax.fori_loop(jnp.int32(0), jnp.int32(GB), e_body, jnp.int32(0))
                return jnp.int32(0)

            lax.fori_loop(jnp.int32(0), nb.astype(jnp.int32), gb_body,
                          jnp.int32(0))
            return jnp.int32(0)

        lax.fori_loop(jnp.int32(0), nblk.astype(jnp.int32), cb_body,
                      jnp.int32(0))

        def rb_body(rb, _, lo=lo):
            def row_body(ii, _):
                i = rb * 8 + ii
                dvvec = acc[i, pl.ds(HC, 16)] + 1e-16
                y = plsc.bitcast(jnp.int32(0x7EF311C3)
                                 - plsc.bitcast(dvvec, jnp.int32), jnp.float32)
                for _n in range(3):
                    y = y * (2.0 - dvvec * y)
                y = y * (1.0 / H)
                for j in range(C // 16):
                    zacc = jnp.zeros((16,), jnp.float32)
                    for h in range(H):
                        off = (h * (C // 16) + j) * 16
                        zacc = zacc + acc[i, pl.ds(off, 16)] * y[h]
                    zwin[ii, pl.ds(j * 16, 16)] = zacc
                return jnp.int32(0)

            lax.fori_loop(jnp.int32(0), jnp.int32(8), row_body, jnp.int32(0))
            pltpu.sync_copy(zwin, z_h.at[pl.ds(lo + rb * 8, 8)])
            return jnp.int32(0)

        lax.fori_loop(jnp.int32(0), jnp.int32(W // 8), rb_body, jnp.int32(0))
        return jnp.int32(0)

    lax.fori_loop(jnp.int32(0), jnp.int32(NWIN), win_body, jnp.int32(0))


@functools.partial(
    pl.kernel,
    mesh=plsc.VectorSubcoreMesh(core_axis_name="c", subcore_axis_name="s"),
    compiler_params=pltpu.CompilerParams(needs_layout_passes=False),
    out_type=(jax.ShapeDtypeStruct((NZ, C), jnp.float32),
              jax.ShapeDtypeStruct((NTILES * CROW,), jnp.int32)),
    scratch_types=[
        pltpu.VMEM((HC,), jnp.float32),
        pltpu.VMEM((FB,), jnp.int32),
        pltpu.VMEM((FB,), jnp.int32),
        pltpu.VMEM((2 * CB + FB + 32,), jnp.int32),
        pltpu.VMEM((CB,), jnp.int32),
        pltpu.VMEM((CB + 48,), jnp.int32),
        pltpu.VMEM((CB + 48,), jnp.int32),
        pltpu.VMEM((2, GB), jnp.int32),
        pltpu.VMEM((2, GB), jnp.int32),
        pltpu.VMEM((2, GB), jnp.int32),
        pltpu.VMEM((2, GB, HC), jnp.float32),
        pltpu.VMEM((2, GB, HC), jnp.float32),
        pltpu.VMEM((W, AW), jnp.float32),
        pltpu.VMEM((8, C), jnp.float32),
        pltpu.SemaphoreType.DMA,
        pltpu.SemaphoreType.DMA,
    ],
)
def _sc_edge(src_h, dst_h, xl_h, xr_h, att_h, zeros_h, z_h, coarse_h,
             *scratch):
    _sc_body(src_h, dst_h, xl_h, xr_h, att_h, zeros_h, z_h, coarse_h,
             *scratch)


def _tc2_body(z_ref, bias_ref, av_ref, ab_ref, wc_ref, bc_ref, out_ref):
    z = jnp.maximum(z_ref[...] + bias_ref[...], 0.0)
    z0 = z[:, :SUB]
    z1 = z[:, SUB:2 * SUB]
    z2 = z[:, 2 * SUB:]
    s0 = jnp.sum(z0 * av_ref[0:1, :], axis=1, keepdims=True) + ab_ref[0, 0]
    s1 = jnp.sum(z1 * av_ref[1:2, :], axis=1, keepdims=True) + ab_ref[1, 0]
    s2 = jnp.sum(z2 * av_ref[2:3, :], axis=1, keepdims=True) + ab_ref[2, 0]
    s0 = jnp.where(s0 > 0, s0, 0.01 * s0)
    s1 = jnp.where(s1 > 0, s1, 0.01 * s1)
    s2 = jnp.where(s2 > 0, s2, 0.01 * s2)
    m = jnp.maximum(jnp.maximum(s0, s1), s2)
    e0 = jnp.exp(s0 - m)
    e1 = jnp.exp(s1 - m)
    e2 = jnp.exp(s2 - m)
    den = e0 + e1 + e2
    zw = jnp.concatenate([z0 * (e0 / den), z1 * (e1 / den), z2 * (e2 / den)],
                         axis=1)
    out_ref[...] = lax.dot_general(
        zw, wc_ref[...], (((1,), (1,)), ((), ())),
        preferred_element_type=jnp.float32) + bc_ref[...]


def _tc2(z_pre, bias, att_vec, att_bias, Wc, bc):
    br = 1000
    grid = (N // br,)
    return pl.pallas_call(
        _tc2_body,
        grid=grid,
        in_specs=[
            pl.BlockSpec((br, C), lambda i: (i, jnp.int32(0))),
            pl.BlockSpec((1, C), lambda i: (jnp.int32(0), jnp.int32(0))),
            pl.BlockSpec((3, SUB), lambda i: (jnp.int32(0), jnp.int32(0))),
            pl.BlockSpec((3, 1), lambda i: (jnp.int32(0), jnp.int32(0))),
            pl.BlockSpec((OUT, C), lambda i: (jnp.int32(0), jnp.int32(0))),
            pl.BlockSpec((1, OUT), lambda i: (jnp.int32(0), jnp.int32(0))),
        ],
        out_specs=pl.BlockSpec((br, OUT), lambda i: (i, jnp.int32(0))),
        out_shape=jax.ShapeDtypeStruct((N, OUT), jnp.float32),
    )(z_pre, bias, att_vec, att_bias, Wc, bc)


def kernel(x, edge_index, Wl, Wr, att, bias, att_vec, att_bias, Wc, bc):
    x = x.astype(jnp.float32)
    ei = edge_index.astype(jnp.int32)
    ar = jnp.arange(N, dtype=jnp.int32)
    pad = E_PAD - E1
    src = jnp.concatenate([ei[0], ar, jnp.zeros((pad,), jnp.int32)])
    dst = jnp.concatenate([ei[1], ar, jnp.full((pad,), 1 << 20, jnp.int32)])

    xl, xr = _tc1(x, Wl.astype(jnp.float32), Wr.astype(jnp.float32))

    att_flat = att.astype(jnp.float32).reshape(HC)
    zeros = jnp.zeros((W, AW), jnp.float32)
    z_pre, _ = _sc_edge(src, dst, xl, xr, att_flat, zeros)

    out = _tc2(z_pre[:N], bias.astype(jnp.float32).reshape(1, C),
               att_vec.astype(jnp.float32), att_bias.astype(jnp.float32),
               Wc.astype(jnp.float32),
               bc.astype(jnp.float32).reshape(1, OUT))
    return out.astype(jnp.float64)
